# baseline (speedup 1.0000x reference)
import functools

import jax
import jax.numpy as jnp
from jax import lax
from jax.experimental import pallas as pl
from jax.experimental.pallas import tpu as pltpu
from jax.experimental.pallas import tpu_sc as plsc

D3 = 21
DM = 64
NHOUR = 24
BATCH, SEQ = 16384, 200
ROWS_TOTAL = BATCH * SEQ
NW = 32
ROWS_PER_W = ROWS_TOTAL // NW
SUB = 128
NSUB = 4
RPC = SUB * NSUB
NCHUNK = ROWS_PER_W // RPC
BLK_PER_W = ROWS_PER_W // SUB


def _table_body(ses_ref, hr_ref, w_ref, b_ref, out_ref):
    h = lax.broadcasted_iota(jnp.int32, (NHOUR, D3), 0)
    r0 = jnp.broadcast_to(ses_ref[0:1, :], (NHOUR, D3))
    r1 = jnp.broadcast_to(ses_ref[1:2, :], (NHOUR, D3))
    r2 = jnp.broadcast_to(ses_ref[2:3, :], (NHOUR, D3))
    ses = jnp.where(h < 8, r0, jnp.where(h < 16, r1, r2))
    ws = w_ref[:, :D3]
    wh = w_ref[:, D3:]
    t = lax.dot_general(ses, ws, (((1,), (1,)), ((), ())),
                        preferred_element_type=jnp.float32)
    t = t + lax.dot_general(hr_ref[...], wh, (((1,), (1,)), ((), ())),
                            preferred_element_type=jnp.float32)
    out_ref[...] = t + b_ref[...]


_table_tc = pl.pallas_call(
    _table_body,
    out_shape=jax.ShapeDtypeStruct((NHOUR, DM), jnp.float32),
)


_mesh = plsc.VectorSubcoreMesh(core_axis_name="c", subcore_axis_name="s")


@functools.partial(
    pl.kernel,
    mesh=_mesh,
    out_type=jax.ShapeDtypeStruct((ROWS_TOTAL, DM), jnp.float32),
    scratch_types=[
        pltpu.VMEM((NSUB, SUB), jnp.int32),
        pltpu.VMEM((NSUB, SUB), jnp.int32),
        pltpu.VMEM((RPC, DM), jnp.float32),
        pltpu.VMEM((RPC, DM), jnp.float32),
        pltpu.SemaphoreType.DMA,
        pltpu.SemaphoreType.DMA,
        pltpu.SemaphoreType.DMA,
        pltpu.SemaphoreType.DMA,
        pltpu.SemaphoreType.DMA,
        pltpu.SemaphoreType.DMA,
    ],
    compiler_params=pltpu.CompilerParams(use_tc_tiling_on_sc=False,
                                         needs_layout_passes=False),
)
def _expand_sc(table_hbm, hour_hbm, out_hbm,
               idx0, idx1, rows0, rows1, si0, si1, sg0, sg1, sw0, sw1):
    idx = [idx0, idx1]
    rows = [rows0, rows1]
    si = [si0, si1]
    sg = [sg0, sg1]
    sw = [sw0, sw1]

    wid = lax.axis_index("s") * 2 + lax.axis_index("c")
    row_base = wid * ROWS_PER_W
    blk_base = wid * BLK_PER_W

    def fire_idx(g, q):
        pltpu.async_copy(hour_hbm.at[pl.ds(blk_base + g * NSUB, NSUB)],
                         idx[q], si[q])

    def wait_idx(q):
        pltpu.make_async_copy(hour_hbm.at[pl.ds(blk_base, NSUB)],
                              idx[q], si[q]).wait()

    def fire_wb(g, p):
        pltpu.async_copy(rows[p],
                         out_hbm.at[pl.ds(row_base + g * RPC, RPC)],
                         sw[p])

    def wait_wb(p):
        pltpu.make_async_copy(rows[p], out_hbm.at[pl.ds(0, RPC)],
                              sw[p]).wait()

    def do_chunk(g, u, fire_next, wait_prev_wb):
        wait_idx(u)
        if wait_prev_wb:
            wait_wb(u)
        for j in range(NSUB):
            pltpu.async_copy(table_hbm.at[idx[u].at[j]],
                             rows[u].at[pl.ds(j * SUB, SUB)], sg[u])
        for j in range(NSUB):
            pltpu.make_async_copy(table_hbm.at[idx[u].at[j]],
                                  rows[u].at[pl.ds(j * SUB, SUB)],
                                  sg[u]).wait()
        if fire_next:
            fire_idx(g + 2, u)
        fire_wb(g, u)

    fire_idx(0, 0)
    fire_idx(1, 1)
    do_chunk(0, 0, True, False)
    do_chunk(1, 1, True, False)

    def body(k, carry):
        for u in range(2):
            do_chunk(2 * k + u, u, True, True)
        return carry

    lax.fori_loop(1, NCHUNK // 2 - 1, body, 0)

    do_chunk(NCHUNK - 2, 0, False, True)
    do_chunk(NCHUNK - 1, 1, False, True)
    wait_wb(0)
    wait_wb(1)


def kernel(hour, session_emb, hour_emb, W, b):
    table = _table_tc(session_emb, hour_emb, W, b.reshape(1, DM))
    hour_blk = hour.astype(jnp.int32).reshape(ROWS_TOTAL // SUB, SUB)
    out = _expand_sc(table, hour_blk)
    return out.reshape(BATCH, SEQ, DM)


# speedup vs baseline: 3.7219x; 3.7219x over previous
import functools

import jax
import jax.numpy as jnp
from jax import lax
from jax.experimental import pallas as pl
from jax.experimental.pallas import tpu as pltpu
from jax.experimental.pallas import tpu_sc as plsc

D3 = 21
DM = 64
NHOUR = 24
BATCH, SEQ = 16384, 200
ROWS_TOTAL = BATCH * SEQ
L = 16
NW = 32
ROWS_PER_W = ROWS_TOTAL // NW
RPC = 800
NCHUNK = ROWS_PER_W // RPC
WPC = RPC * DM


def _table_body(ses_ref, hr_ref, w_ref, b_ref, out_ref):
    h = lax.broadcasted_iota(jnp.int32, (NHOUR, D3), 0)
    r0 = jnp.broadcast_to(ses_ref[0:1, :], (NHOUR, D3))
    r1 = jnp.broadcast_to(ses_ref[1:2, :], (NHOUR, D3))
    r2 = jnp.broadcast_to(ses_ref[2:3, :], (NHOUR, D3))
    ses = jnp.where(h < 8, r0, jnp.where(h < 16, r1, r2))
    ws = w_ref[:, :D3]
    wh = w_ref[:, D3:]
    t = lax.dot_general(ses, ws, (((1,), (1,)), ((), ())),
                        preferred_element_type=jnp.float32)
    t = t + lax.dot_general(hr_ref[...], wh, (((1,), (1,)), ((), ())),
                            preferred_element_type=jnp.float32)
    out_ref[...] = t + b_ref[...]


_table_tc = pl.pallas_call(
    _table_body,
    out_shape=jax.ShapeDtypeStruct((NHOUR, DM), jnp.float32),
)


_mesh = plsc.VectorSubcoreMesh(core_axis_name="c", subcore_axis_name="s")


@functools.partial(
    pl.kernel,
    mesh=_mesh,
    out_type=jax.ShapeDtypeStruct((ROWS_TOTAL * DM,), jnp.float32),
    scratch_types=[
        pltpu.VMEM((NHOUR * DM,), jnp.float32),
        pltpu.VMEM((RPC,), jnp.int32),
        pltpu.VMEM((RPC,), jnp.int32),
        pltpu.VMEM((WPC,), jnp.float32),
        pltpu.VMEM((WPC,), jnp.float32),
        pltpu.SemaphoreType.DMA,
        pltpu.SemaphoreType.DMA,
        pltpu.SemaphoreType.DMA,
        pltpu.SemaphoreType.DMA,
    ],
    compiler_params=pltpu.CompilerParams(use_tc_tiling_on_sc=False,
                                         needs_layout_passes=False),
)
def _expand_sc(table_hbm, hour_hbm, out_hbm,
               table_v, idx0, idx1, rows0, rows1, si0, si1, sw0, sw1):
    idx = [idx0, idx1]
    rows = [rows0, rows1]
    si = [si0, si1]
    sw = [sw0, sw1]

    wid = lax.axis_index("s") * 2 + lax.axis_index("c")
    row_base = wid * ROWS_PER_W

    iota = lax.iota(jnp.int32, L)

    def fire_idx(g, q):
        pltpu.async_copy(hour_hbm.at[pl.ds(row_base + g * RPC, RPC)],
                         idx[q], si[q])

    def wait_idx(q):
        pltpu.make_async_copy(hour_hbm.at[pl.ds(row_base, RPC)],
                              idx[q], si[q]).wait()

    def fire_wb(g, p):
        pltpu.async_copy(rows[p],
                         out_hbm.at[pl.ds((row_base + g * RPC) * DM, WPC)],
                         sw[p])

    def wait_wb(p):
        pltpu.make_async_copy(rows[p], out_hbm.at[pl.ds(0, WPC)],
                              sw[p]).wait()

    def compute(p, q):
        @plsc.parallel_loop(0, RPC // L, unroll=2)
        def group(t):
            s_vec = idx[q][pl.ds(t * L, L)]
            off = s_vec * DM
            for i in range(L):
                base = off[i]
                r0 = (t * L + i) * DM
                for cb in range(DM // L):
                    rows[p][pl.ds(r0 + cb * L, L)] = (
                        table_v[pl.ds(base + cb * L, L)])

    pltpu.sync_copy(table_hbm, table_v)

    fire_idx(0, 0)
    fire_idx(1, 1)
    wait_idx(0)
    compute(0, 0)
    fire_wb(0, 0)
    fire_idx(2, 0)
    wait_idx(1)
    compute(1, 1)
    fire_wb(1, 1)
    fire_idx(3, 1)

    def body(k, carry):
        for u in range(2):
            g = 2 * k + u
            wait_idx(u)
            wait_wb(u)
            compute(u, u)
            fire_wb(g, u)
            fire_idx(g + 2, u)
        return carry

    lax.fori_loop(1, NCHUNK // 2 - 1, body, 0)

    for g in (NCHUNK - 2, NCHUNK - 1):
        u = g % 2
        wait_idx(u)
        wait_wb(u)
        compute(u, u)
        fire_wb(g, u)
    wait_wb(0)
    wait_wb(1)


def kernel(hour, session_emb, hour_emb, W, b):
    table = _table_tc(session_emb, hour_emb, W, b.reshape(1, DM))
    hour_flat = hour.astype(jnp.int32).reshape(ROWS_TOTAL)
    out = _expand_sc(table.reshape(NHOUR * DM), hour_flat)
    return out.reshape(BATCH, SEQ, DM)
